# baseline (speedup 1.0000x reference)
import functools
import jax
import jax.numpy as jnp
from jax import lax
from jax.experimental import pallas as pl
from jax.experimental.pallas import tpu as pltpu
from jax.experimental.pallas import tpu_sc as plsc

B = 4096
ND = 13
F = 26
V = 100000
D = 64

NC = 2
NS = 16
NW = NC * NS
R = F * D
NSLAB = R // 8
VB = 11136
NFULL = 8
VMID = 10880
VLAST = 32
NBIN = 9
NVEC = B // 16
PCAP = 4240


def _emb_gather_t(tables_t, tail32, idx_flat):
    mesh = plsc.VectorSubcoreMesh(core_axis_name="c", subcore_axis_name="s")

    @functools.partial(
        pl.kernel,
        mesh=mesh,
        compiler_params=pltpu.CompilerParams(
            use_tc_tiling_on_sc=True, needs_layout_passes=False
        ),
        out_type=jax.ShapeDtypeStruct((R, B), jnp.float32),
        scratch_types=[
            pltpu.VMEM((B,), jnp.int32),
            pltpu.VMEM((8, VB), jnp.float32),
            pltpu.VMEM((8, B), jnp.float32),
            pltpu.VMEM((PCAP,), jnp.int32),
            pltpu.VMEM((16,), jnp.int32),
        ],
    )
    def k(t_hbm, tail_hbm, idx_hbm, out_hbm, idx_v, slab_v, out_v, packed_v,
          cnt_v):
        wid = lax.axis_index("s") * NC + lax.axis_index("c")
        lane = jnp.arange(16, dtype=jnp.int32)
        ones = jnp.ones((16,), dtype=jnp.int32)
        lo = (13 * wid) // 2
        hi = (13 * (wid + 1)) // 2

        @pl.loop(lo // 8, (hi + 7) // 8)
        def _field(f):
            pltpu.sync_copy(idx_hbm.at[pl.ds(f * B, B)], idx_v)

            cnt_v[...] = jnp.zeros((16,), dtype=jnp.int32)

            @pl.loop(0, NVEC)
            def _hist(i):
                cid = idx_v[pl.ds(i * 16, 16)] // VB
                plsc.addupdate_scatter(cnt_v, [cid], ones)

            cnt16 = cnt_v[...]
            rnd = (cnt16 + 15) & (-16)
            off16 = plsc.cumsum(rnd) - rnd
            offs = [off16[b] for b in range(NBIN)]
            cnts = [cnt16[b] for b in range(NBIN)]

            @pl.loop(0, NVEC, init_carry=tuple(offs))
            def _place(i, fills):
                idx = idx_v[pl.ds(i * 16, 16)]
                cid = idx // VB
                p = (idx << 12) | (lane + i * 16)
                nxt = []
                for b in range(NBIN):
                    m = cid == b
                    plsc.store_compressed(
                        packed_v.at[pl.ds(fills[b], 16)], p, mask=m
                    )
                    c = jnp.max(plsc.all_reduce_population_count(m))
                    nxt.append(fills[b] + c)
                return tuple(nxt)

            def resolve_bin(b, v0, width):
                b_off, b_cnt = offs[b], cnts[b]
                nv = (b_cnt + 15) >> 4

                @pl.loop(0, nv)
                def _v(j):
                    p = packed_v[pl.ds(b_off + j * 16, 16)]
                    idx = p >> 12
                    li = idx - v0
                    pos = p & 4095
                    m = (lane < (b_cnt - j * 16)) & (li >= 0) & (li < width)
                    for dr in range(8):
                        dr_vec = jnp.full((16,), dr, dtype=jnp.int32)
                        g = plsc.load_gather(slab_v, [dr_vec, li], mask=m)
                        plsc.store_scatter(out_v, [dr_vec, pos], g, mask=m)

            @pl.loop(jnp.maximum(lo, 8 * f), jnp.minimum(hi, 8 * f + 8))
            def _slab(s):
                for c in range(NFULL):
                    pltpu.sync_copy(
                        t_hbm.at[pl.ds(s * 8, 8), pl.ds(c * VB, VB)], slab_v
                    )
                    resolve_bin(c, c * VB, VB)

                pltpu.sync_copy(
                    t_hbm.at[pl.ds(s * 8, 8), pl.ds(NFULL * VB, VMID)],
                    slab_v.at[:, pl.ds(0, VMID)],
                )
                resolve_bin(NBIN - 1, NFULL * VB, VMID)

                pltpu.sync_copy(
                    tail_hbm.at[pl.ds(s * 8, 8)], slab_v.at[:, pl.ds(0, 128)]
                )
                resolve_bin(NBIN - 1, NFULL * VB + VMID, VLAST)

                pltpu.sync_copy(out_v, out_hbm.at[pl.ds(s * 8, 8)])

    return k(tables_t, tail32, idx_flat)


def kernel(X_num, X_cat, tables):
    tables_t = tables.transpose(0, 2, 1).reshape(R, V)
    tail32 = jnp.pad(tables_t[:, NFULL * VB + VMID :], ((0, 0), (0, 128 - VLAST)))
    idx_flat = X_cat.T.reshape(F * B)
    emb_t = _emb_gather_t(tables_t, tail32, idx_flat)
    return jnp.concatenate([X_num, emb_t.T], axis=1)


# speedup vs baseline: 1.3011x; 1.3011x over previous
import functools
import jax
import jax.numpy as jnp
from jax import lax
from jax.experimental import pallas as pl
from jax.experimental.pallas import tpu as pltpu
from jax.experimental.pallas import tpu_sc as plsc

B = 4096
ND = 13
F = 26
V = 100000
D = 64

NC = 2
NS = 16
NW = NC * NS
R = F * D
NSLAB = R // 8
VB = 5504
NFULL = 18
VMID = 896
VLAST = 32
NBIN = 19
NVEC = B // 16
PCAP = 4384


def _emb_gather_t(tables_t, tail32, idx_flat):
    mesh = plsc.VectorSubcoreMesh(core_axis_name="c", subcore_axis_name="s")

    @functools.partial(
        pl.kernel,
        mesh=mesh,
        compiler_params=pltpu.CompilerParams(
            use_tc_tiling_on_sc=True, needs_layout_passes=False
        ),
        out_type=jax.ShapeDtypeStruct((R, B), jnp.float32),
        scratch_types=[
            pltpu.VMEM((B,), jnp.int32),
            pltpu.VMEM((16, VB), jnp.float32),
            pltpu.VMEM((8, B), jnp.float32),
            pltpu.VMEM((PCAP,), jnp.int32),
            pltpu.VMEM((32,), jnp.int32),
            pltpu.SemaphoreType.DMA,
        ],
    )
    def k(t_hbm, tail_hbm, idx_hbm, out_hbm, idx_v, slab_v, out_v, packed_v,
          cnt_v, sem):
        wid = lax.axis_index("s") * NC + lax.axis_index("c")
        lane = jnp.arange(16, dtype=jnp.int32)
        ones = jnp.ones((16,), dtype=jnp.int32)
        zeros16 = jnp.zeros((16,), dtype=jnp.int32)
        lo = (13 * wid) // 2
        hi = (13 * (wid + 1)) // 2

        def full_cp(s, c):
            return pltpu.make_async_copy(
                t_hbm.at[pl.ds(s * 8, 8), pl.ds(c * VB, VB)],
                slab_v.at[pl.ds((c % 2) * 8, 8)],
                sem,
            )

        def mid_cp(s):
            return pltpu.make_async_copy(
                t_hbm.at[pl.ds(s * 8, 8), pl.ds(NFULL * VB, VMID)],
                slab_v.at[pl.ds(0, 8), pl.ds(0, VMID)],
                sem,
            )

        def last_cp(s):
            return pltpu.make_async_copy(
                tail_hbm.at[pl.ds(s * 8, 8)],
                slab_v.at[pl.ds(8, 8), pl.ds(0, 128)],
                sem,
            )

        @pl.loop(lo // 8, (hi + 7) // 8)
        def _field(f):
            pltpu.sync_copy(idx_hbm.at[pl.ds(f * B, B)], idx_v)

            cnt_v[pl.ds(0, 16)] = zeros16
            cnt_v[pl.ds(16, 16)] = zeros16

            @pl.loop(0, NVEC)
            def _hist(i):
                cid = idx_v[pl.ds(i * 16, 16)] // VB
                plsc.addupdate_scatter(
                    cnt_v, [jnp.minimum(cid, 15)], ones, mask=cid < 16
                )
                plsc.addupdate_scatter(
                    cnt_v, [jnp.maximum(cid, 16)], ones, mask=cid >= 16
                )

            cnt_lo = cnt_v[pl.ds(0, 16)]
            cnt_hi = cnt_v[pl.ds(16, 16)]
            rnd_lo = (cnt_lo + 15) & (-16)
            rnd_hi = (cnt_hi + 15) & (-16)
            off_lo = plsc.cumsum(rnd_lo) - rnd_lo
            off_hi = plsc.cumsum(rnd_hi) - rnd_hi + jnp.sum(rnd_lo)
            offs = [off_lo[b] for b in range(16)] + [
                off_hi[b] for b in range(NBIN - 16)
            ]
            cnts = [cnt_lo[b] for b in range(16)] + [
                cnt_hi[b] for b in range(NBIN - 16)
            ]

            @pl.loop(0, NVEC, init_carry=tuple(offs))
            def _place(i, fills):
                idx = idx_v[pl.ds(i * 16, 16)]
                cid = idx // VB
                p = (idx << 12) | (lane + i * 16)
                nxt = []
                for b in range(NBIN):
                    m = cid == b
                    plsc.store_compressed(
                        packed_v.at[pl.ds(fills[b], 16)], p, mask=m
                    )
                    c = jnp.max(plsc.all_reduce_population_count(m))
                    nxt.append(fills[b] + c)
                return tuple(nxt)

            def resolve_bin(b, v0, width, half):
                b_off, b_cnt = offs[b], cnts[b]
                nv = (b_cnt + 15) >> 4
                hb = half * 8

                @pl.loop(0, nv)
                def _v(j):
                    p = packed_v[pl.ds(b_off + j * 16, 16)]
                    idx = p >> 12
                    li = idx - v0
                    pos = p & 4095
                    m = (lane < (b_cnt - j * 16)) & (li >= 0) & (li < width)
                    for dr in range(8):
                        src = jnp.full((16,), hb + dr, dtype=jnp.int32)
                        dst = jnp.full((16,), dr, dtype=jnp.int32)
                        g = plsc.load_gather(slab_v, [src, li], mask=m)
                        plsc.store_scatter(out_v, [dst, pos], g, mask=m)

            @pl.loop(jnp.maximum(lo, 8 * f), jnp.minimum(hi, 8 * f + 8))
            def _slab(s):
                def cp(c):
                    if c < NFULL:
                        return full_cp(s, c)
                    if c == NFULL:
                        return mid_cp(s)
                    return last_cp(s)

                def resolve(c):
                    if c < NFULL:
                        resolve_bin(c, c * VB, VB, c % 2)
                    elif c == NFULL:
                        resolve_bin(NBIN - 1, NFULL * VB, VMID, 0)
                    else:
                        resolve_bin(NBIN - 1, NFULL * VB + VMID, VLAST, 1)

                cp(0).start()
                cp(1).start()
                for c in range(NFULL + 2):
                    cp(c).wait()
                    resolve(c)
                    if c + 2 < NFULL + 2:
                        cp(c + 2).start()

                pltpu.sync_copy(out_v, out_hbm.at[pl.ds(s * 8, 8)])

    return k(tables_t, tail32, idx_flat)


def kernel(X_num, X_cat, tables):
    tables_t = tables.transpose(0, 2, 1).reshape(R, V)
    tail32 = jnp.pad(tables_t[:, NFULL * VB + VMID :], ((0, 0), (0, 128 - VLAST)))
    idx_flat = X_cat.T.reshape(F * B)
    emb_t = _emb_gather_t(tables_t, tail32, idx_flat)
    return jnp.concatenate([X_num, emb_t.T], axis=1)


# speedup vs baseline: 1.3148x; 1.0106x over previous
import functools
import jax
import jax.numpy as jnp
from jax import lax
from jax.experimental import pallas as pl
from jax.experimental.pallas import tpu as pltpu
from jax.experimental.pallas import tpu_sc as plsc

B = 4096
ND = 13
F = 26
V = 100000
D = 64

NC = 2
NS = 16
NW = NC * NS
R = F * D
NSLAB = R // 8
VB = 5504
NFULL = 18
VMID = 896
VLAST = 32
NBIN = 19
NVEC = B // 16
PCAP = 4384


def _emb_gather_t(tables_t, tail32, idx_flat):
    mesh = plsc.VectorSubcoreMesh(core_axis_name="c", subcore_axis_name="s")

    @functools.partial(
        pl.kernel,
        mesh=mesh,
        compiler_params=pltpu.CompilerParams(
            use_tc_tiling_on_sc=True, needs_layout_passes=False
        ),
        out_type=jax.ShapeDtypeStruct((R, B), jnp.float32),
        scratch_types=[
            pltpu.VMEM((B,), jnp.int32),
            pltpu.VMEM((16, VB), jnp.float32),
            pltpu.VMEM((8, B), jnp.float32),
            pltpu.VMEM((PCAP,), jnp.int32),
            pltpu.VMEM((32,), jnp.int32),
            pltpu.SemaphoreType.DMA,
        ],
    )
    def k(t_hbm, tail_hbm, idx_hbm, out_hbm, idx_v, slab_v, out_v, packed_v,
          cnt_v, sem):
        wid = lax.axis_index("s") * NC + lax.axis_index("c")
        lane = jnp.arange(16, dtype=jnp.int32)
        ones = jnp.ones((16,), dtype=jnp.int32)
        zeros16 = jnp.zeros((16,), dtype=jnp.int32)
        lo = (13 * wid) // 2
        hi = (13 * (wid + 1)) // 2

        def full_cp(s, c):
            return pltpu.make_async_copy(
                t_hbm.at[pl.ds(s * 8, 8), pl.ds(c * VB, VB)],
                slab_v.at[pl.ds((c % 2) * 8, 8)],
                sem,
            )

        def mid_cp(s):
            return pltpu.make_async_copy(
                t_hbm.at[pl.ds(s * 8, 8), pl.ds(NFULL * VB, VMID)],
                slab_v.at[pl.ds(0, 8), pl.ds(0, VMID)],
                sem,
            )

        def last_cp(s):
            return pltpu.make_async_copy(
                tail_hbm.at[pl.ds(s * 8, 8)],
                slab_v.at[pl.ds(8, 8), pl.ds(0, 128)],
                sem,
            )

        @pl.loop(lo // 8, (hi + 7) // 8)
        def _field(f):
            pltpu.sync_copy(idx_hbm.at[pl.ds(f * B, B)], idx_v)

            cnt_v[pl.ds(0, 16)] = zeros16
            cnt_v[pl.ds(16, 16)] = zeros16

            @pl.loop(0, NVEC)
            def _hist(i):
                cid = idx_v[pl.ds(i * 16, 16)] // VB
                plsc.addupdate_scatter(
                    cnt_v, [jnp.minimum(cid, 15)], ones, mask=cid < 16
                )
                plsc.addupdate_scatter(
                    cnt_v, [jnp.maximum(cid, 16)], ones, mask=cid >= 16
                )

            cnt_lo = cnt_v[pl.ds(0, 16)]
            cnt_hi = cnt_v[pl.ds(16, 16)]
            rnd_lo = (cnt_lo + 15) & (-16)
            rnd_hi = (cnt_hi + 15) & (-16)
            off_lo = plsc.cumsum(rnd_lo) - rnd_lo
            off_hi = plsc.cumsum(rnd_hi) - rnd_hi + jnp.sum(rnd_lo)
            offs = [off_lo[b] for b in range(16)] + [
                off_hi[b] for b in range(NBIN - 16)
            ]
            cnts = [cnt_lo[b] for b in range(16)] + [
                cnt_hi[b] for b in range(NBIN - 16)
            ]

            @pl.loop(0, NVEC, init_carry=tuple(offs))
            def _place(i, fills):
                idx = idx_v[pl.ds(i * 16, 16)]
                cid = idx // VB
                p = (idx << 12) | (lane + i * 16)
                nxt = []
                for b in range(NBIN):
                    m = cid == b
                    plsc.store_compressed(
                        packed_v.at[pl.ds(fills[b], 16)], p, mask=m
                    )
                    c = jnp.max(plsc.all_reduce_population_count(m))
                    nxt.append(fills[b] + c)
                return tuple(nxt)

            def resolve_bin(b, v0, width, half):
                b_off, b_cnt = offs[b], cnts[b]
                nv = (b_cnt + 15) >> 4
                hb = half * 8

                @pl.loop(0, nv)
                def _v(j):
                    p = packed_v[pl.ds(b_off + j * 16, 16)]
                    idx = p >> 12
                    li = idx - v0
                    pos = p & 4095
                    m = (lane < (b_cnt - j * 16)) & (li >= 0) & (li < width)
                    for dr in range(8):
                        src = jnp.full((16,), hb + dr, dtype=jnp.int32)
                        dst = jnp.full((16,), dr, dtype=jnp.int32)
                        g = plsc.load_gather(slab_v, [src, li], mask=m)
                        plsc.store_scatter(out_v, [dst, pos], g, mask=m)

            slab_lo = jnp.maximum(lo, 8 * f)
            slab_hi = jnp.minimum(hi, 8 * f + 8)

            @pl.loop(slab_lo, slab_hi)
            def _slab(s):
                def cp(c):
                    if c < NFULL:
                        return full_cp(s, c)
                    if c == NFULL:
                        return mid_cp(s)
                    return last_cp(s)

                def resolve(c):
                    if c < NFULL:
                        resolve_bin(c, c * VB, VB, c % 2)
                    elif c == NFULL:
                        resolve_bin(NBIN - 1, NFULL * VB, VMID, 0)
                    else:
                        resolve_bin(NBIN - 1, NFULL * VB + VMID, VLAST, 1)

                @pl.when(s == slab_lo)
                def _():
                    cp(0).start()
                    cp(1).start()

                for c in range(NFULL + 2):
                    cp(c).wait()
                    resolve(c)
                    if c + 2 < NFULL + 2:
                        cp(c + 2).start()

                @pl.when(s + 1 < slab_hi)
                def _():
                    pltpu.make_async_copy(
                        t_hbm.at[pl.ds(s * 8 + 8, 8), pl.ds(0, VB)],
                        slab_v.at[pl.ds(0, 8)],
                        sem,
                    ).start()
                    pltpu.make_async_copy(
                        t_hbm.at[pl.ds(s * 8 + 8, 8), pl.ds(VB, VB)],
                        slab_v.at[pl.ds(8, 8)],
                        sem,
                    ).start()

                pltpu.sync_copy(out_v, out_hbm.at[pl.ds(s * 8, 8)])

    return k(tables_t, tail32, idx_flat)


def kernel(X_num, X_cat, tables):
    tables_t = tables.transpose(0, 2, 1).reshape(R, V)
    tail32 = jnp.pad(tables_t[:, NFULL * VB + VMID :], ((0, 0), (0, 128 - VLAST)))
    idx_flat = X_cat.T.reshape(F * B)
    emb_t = _emb_gather_t(tables_t, tail32, idx_flat)
    return jnp.concatenate([X_num, emb_t.T], axis=1)
